# baseline (speedup 1.0000x reference)
import functools

import jax
import jax.numpy as jnp
from jax import lax
from jax.experimental import pallas as pl
from jax.experimental.pallas import tpu as pltpu
from jax.experimental.pallas import tpu_sc as plsc

_N = 16384
_C = 1000
_D = 4096
_ALPHA = 1.0

_NC = 2
_NS = 16
_NW = _NC * _NS
_SPW = _N // _NW
_CH = 4
_NCH = _SPW // _CH
_UNROLL = 4


def _body(x_hbm, cls_hbm, cen_hbm, out_hbm, idx_v, xbuf, cbuf, out_v,
          sem_x, sem_c):
    wid = lax.axis_index("s") * _NC + lax.axis_index("c")
    base = wid * _SPW

    pltpu.sync_copy(cls_hbm.at[pl.ds(wid * _NCH, _NCH)], idx_v)

    lane = lax.iota(jnp.int32, 16)
    inv_d = jnp.float32(_ALPHA / _D)

    def group_body(g, _):
        out_vec = jnp.zeros((16,), jnp.float32)
        for k in range(4):
            j = g * 4 + k
            cp_x = pltpu.async_copy(
                x_hbm.at[pl.ds(base + j * _CH, _CH)], xbuf, sem_x)
            cp_c = pltpu.async_copy(cen_hbm.at[idx_v.at[j]], cbuf, sem_c)
            cp_x.wait()
            cp_c.wait()
            for s in range(_CH):
                def dstep(i, accs, s=s):
                    d0 = i * (16 * _UNROLL)
                    new = []
                    for u in range(_UNROLL):
                        xa = xbuf[s, pl.ds(d0 + 16 * u, 16)]
                        ca = cbuf[s, pl.ds(d0 + 16 * u, 16)]
                        df = xa - ca
                        new.append(accs[u] + df * df)
                    return tuple(new)

                accs = lax.fori_loop(
                    0, _D // (16 * _UNROLL), dstep,
                    tuple(jnp.zeros((16,), jnp.float32)
                          for _ in range(_UNROLL)))
                total = accs[0]
                for u in range(1, _UNROLL):
                    total = total + accs[u]
                t = plsc.cumsum(total)[15] * inv_d
                out_vec = jnp.where(lane == (k * _CH + s), t, out_vec)
        out_v[pl.ds(g * 16, 16)] = out_vec
        return 0

    lax.fori_loop(0, _SPW // 16, group_body, 0)
    pltpu.sync_copy(out_v, out_hbm.at[pl.ds(base, _SPW)])


@jax.jit
def _loss_center_sc(x, cls2, centers):
    mesh = plsc.VectorSubcoreMesh(core_axis_name="c", subcore_axis_name="s")
    return pl.kernel(
        _body,
        out_type=jax.ShapeDtypeStruct((_N,), jnp.float32),
        mesh=mesh,
        scratch_types=[
            pltpu.VMEM((_NCH, _CH), jnp.int32),
            pltpu.VMEM((_CH, _D), jnp.float32),
            pltpu.VMEM((_CH, _D), jnp.float32),
            pltpu.VMEM((_SPW,), jnp.float32),
            pltpu.SemaphoreType.DMA,
            pltpu.SemaphoreType.DMA,
        ],
        compiler_params=pltpu.CompilerParams(needs_layout_passes=False),
        name="loss_center_sc",
    )(x, cls2, centers)


def kernel(x, classes, centers):
    cls2 = jnp.asarray(classes, jnp.int32).reshape(_N // _CH, _CH)
    return _loss_center_sc(x, cls2, centers)


# speedup vs baseline: 1.5293x; 1.5293x over previous
import functools

import jax
import jax.numpy as jnp
from jax import lax
from jax.experimental import pallas as pl
from jax.experimental.pallas import tpu as pltpu
from jax.experimental.pallas import tpu_sc as plsc

_N = 16384
_C = 1000
_D = 4096
_ALPHA = 1.0

_NC = 2
_NS = 16
_NW = _NC * _NS
_SPW = _N // _NW
_CH = 4
_NCH = _SPW // _CH
_UNROLL = 4


def _body(x_hbm, cls_hbm, cen_hbm, out_hbm, idx_v, xb0, xb1, cb0, cb1,
          out_v, sx0, sx1, sc0, sc1):
    wid = lax.axis_index("s") * _NC + lax.axis_index("c")
    base = wid * _SPW

    pltpu.sync_copy(cls_hbm.at[pl.ds(wid * _NCH, _NCH)], idx_v)

    xbufs = (xb0, xb1)
    cbufs = (cb0, cb1)
    sxs = (sx0, sx1)
    scs = (sc0, sc1)

    def fire(j, b):
        pltpu.async_copy(x_hbm.at[pl.ds(base + j * _CH, _CH)], xbufs[b],
                         sxs[b])
        pltpu.async_copy(cen_hbm.at[idx_v.at[j]], cbufs[b], scs[b])

    def drain(b):
        pltpu.make_async_copy(x_hbm.at[pl.ds(0, _CH)], xbufs[b],
                              sxs[b]).wait()
        pltpu.make_async_copy(cen_hbm.at[pl.ds(0, _CH)], cbufs[b],
                              scs[b]).wait()

    lane = lax.iota(jnp.int32, 16)
    inv_d = jnp.float32(_ALPHA / _D)
    zeros = tuple(jnp.zeros((16,), jnp.float32) for _ in range(_UNROLL))

    fire(0, 0)

    def group_body(g, _):
        out_vec = jnp.zeros((16,), jnp.float32)
        for k in range(4):
            j = g * 4 + k
            b = k % 2
            drain(b)

            @pl.when(j + 1 < _NCH)
            def _():
                fire(j + 1, (k + 1) % 2)

            for s in range(_CH):
                @plsc.parallel_loop(0, _D // 16, step=_UNROLL,
                                    unroll=2, carry=zeros)
                def accs(i, acc, s=s, b=b):
                    new = []
                    for u in range(_UNROLL):
                        xa = xbufs[b][s, pl.ds((i + u) * 16, 16)]
                        ca = cbufs[b][s, pl.ds((i + u) * 16, 16)]
                        df = xa - ca
                        new.append(acc[u] + df * df)
                    return tuple(new)

                total = accs[0]
                for u in range(1, _UNROLL):
                    total = total + accs[u]
                t = plsc.cumsum(total)[15] * inv_d
                out_vec = jnp.where(lane == (k * _CH + s), t, out_vec)
        out_v[pl.ds(g * 16, 16)] = out_vec
        return 0

    lax.fori_loop(0, _SPW // 16, group_body, 0)
    pltpu.sync_copy(out_v, out_hbm.at[pl.ds(base, _SPW)])


@jax.jit
def _loss_center_sc(x, cls2, centers):
    mesh = plsc.VectorSubcoreMesh(core_axis_name="c", subcore_axis_name="s")
    return pl.kernel(
        _body,
        out_type=jax.ShapeDtypeStruct((_N,), jnp.float32),
        mesh=mesh,
        scratch_types=[
            pltpu.VMEM((_NCH, _CH), jnp.int32),
            pltpu.VMEM((_CH, _D), jnp.float32),
            pltpu.VMEM((_CH, _D), jnp.float32),
            pltpu.VMEM((_CH, _D), jnp.float32),
            pltpu.VMEM((_CH, _D), jnp.float32),
            pltpu.VMEM((_SPW,), jnp.float32),
            pltpu.SemaphoreType.DMA,
            pltpu.SemaphoreType.DMA,
            pltpu.SemaphoreType.DMA,
            pltpu.SemaphoreType.DMA,
        ],
        compiler_params=pltpu.CompilerParams(needs_layout_passes=False),
        name="loss_center_sc",
    )(x, cls2, centers)


def kernel(x, classes, centers):
    cls2 = jnp.asarray(classes, jnp.int32).reshape(_N // _CH, _CH)
    return _loss_center_sc(x, cls2, centers)
